# baseline (speedup 1.0000x reference)
import functools

import jax
import jax.numpy as jnp
from jax.experimental import pallas as pl
from jax.experimental.pallas import tpu as pltpu

_THRESHOLD = 0.3
_TRUE_THRESHOLD = 0.5
_PENALTY_WEIGHT = 0.05

_B, _H, _W, _C = 8, 512, 512, 21
_RC = 16
_NCH = _H // _RC


def _bbox_from_scalars(y_min, y_max, x_min, x_max):
    is_empty = y_max < 0.0
    y0 = jnp.where(is_empty, 0.0, y_min)
    x0 = jnp.where(is_empty, 0.0, x_min)
    y1 = jnp.where(is_empty, 1.0, y_max)
    x1 = jnp.where(is_empty, 1.0, x_max)
    return y0, x0, y1, x1


def _penalty_scalar(py0, px0, py1, px1, ty0, tx0, ty1, tx1):
    pred_area = (py1 - py0 + 1.0) * (px1 - px0 + 1.0)
    true_area = (ty1 - ty0 + 1.0) * (tx1 - tx0 + 1.0)
    area_penalty = jnp.maximum(pred_area - true_area, 0.0) / (true_area + 1.0)
    dy = (py0 + py1) / 2.0 - (ty0 + ty1) / 2.0
    dx = (px0 + px1) / 2.0 - (tx0 + tx1) / 2.0
    center_offset = jnp.sqrt(dy * dy + dx * dx) / 20.0
    return area_penalty + center_offset


def _body(p_ref, t_ref, out_ref, colp_ref, colt_ref, acc_ref):
    b = pl.program_id(0)
    c = pl.program_id(1)
    first = c == 0

    xp = p_ref[0]
    xt = t_ref[0]

    cp = jnp.max(xp, axis=0)
    ct = jnp.max(xt, axis=0)
    colp_ref[...] = jnp.where(first, cp, jnp.maximum(colp_ref[...], cp))
    colt_ref[...] = jnp.where(first, ct, jnp.maximum(colt_ref[...], ct))

    rp = jnp.max(xp, axis=(1, 2)).reshape(1, _RC) > _THRESHOLD
    rt = jnp.max(xt, axis=(1, 2)).reshape(1, _RC) > _TRUE_THRESHOLD
    yidx = jax.lax.broadcasted_iota(jnp.int32, (1, _RC), 1).astype(
        jnp.float32) + (c * _RC).astype(jnp.float32)
    ymin_p = jnp.min(jnp.where(rp, yidx, float(_H)))
    ymax_p = jnp.max(jnp.where(rp, yidx, -1.0))
    ymin_t = jnp.min(jnp.where(rt, yidx, float(_H)))
    ymax_t = jnp.max(jnp.where(rt, yidx, -1.0))

    acc_ref[0] = jnp.where(first, ymin_p, jnp.minimum(acc_ref[0], ymin_p))
    acc_ref[1] = jnp.where(first, ymax_p, jnp.maximum(acc_ref[1], ymax_p))
    acc_ref[2] = jnp.where(first, ymin_t, jnp.minimum(acc_ref[2], ymin_t))
    acc_ref[3] = jnp.where(first, ymax_t, jnp.maximum(acc_ref[3], ymax_t))

    @pl.when(c == _NCH - 1)
    def _finish_image():
        xmap = jax.lax.broadcasted_iota(jnp.int32, (_W, _C), 0).astype(
            jnp.float32)
        cmp_ = colp_ref[...] > _THRESHOLD
        cmt_ = colt_ref[...] > _TRUE_THRESHOLD
        xmin_p = jnp.min(jnp.where(cmp_, xmap, float(_W)))
        xmax_p = jnp.max(jnp.where(cmp_, xmap, -1.0))
        xmin_t = jnp.min(jnp.where(cmt_, xmap, float(_W)))
        xmax_t = jnp.max(jnp.where(cmt_, xmap, -1.0))

        pb = _bbox_from_scalars(acc_ref[0], acc_ref[1], xmin_p, xmax_p)
        tb = _bbox_from_scalars(acc_ref[2], acc_ref[3], xmin_t, xmax_t)
        pen = _penalty_scalar(*pb, *tb)
        psum = jnp.where(b == 0, pen, acc_ref[4] + pen)
        acc_ref[4] = psum

        @pl.when(b == _B - 1)
        def _write_out():
            out_ref[0, 0] = _PENALTY_WEIGHT * psum / float(_B)


@jax.jit
def _run(pred, true):
    out = pl.pallas_call(
        _body,
        grid=(_B, _NCH),
        in_specs=[
            pl.BlockSpec((1, _RC, _W, _C), lambda b, c: (b, c, 0, 0)),
            pl.BlockSpec((1, _RC, _W, _C), lambda b, c: (b, c, 0, 0)),
        ],
        out_specs=pl.BlockSpec(memory_space=pltpu.SMEM),
        out_shape=jax.ShapeDtypeStruct((1, 1), jnp.float32),
        scratch_shapes=[
            pltpu.VMEM((_W, _C), jnp.float32),
            pltpu.VMEM((_W, _C), jnp.float32),
            pltpu.SMEM((8,), jnp.float32),
        ],
        compiler_params=pltpu.CompilerParams(
            dimension_semantics=("arbitrary", "arbitrary"),
        ),
    )(pred, true)
    return out[0, 0]


def kernel(prediction_probs, expected_onehot):
    return _run(prediction_probs, expected_onehot)


# speedup vs baseline: 16.9098x; 16.9098x over previous
import jax
import jax.numpy as jnp
from jax.experimental import pallas as pl
from jax.experimental.pallas import tpu as pltpu

_THRESHOLD = 0.3
_TRUE_THRESHOLD = 0.5
_PENALTY_WEIGHT = 0.05

_B, _H, _W, _C = 8, 512, 512, 21
_RC = 64
_NCH = _H // _RC


def _bbox_from_scalars(y_min, y_max, x_min, x_max):
    is_empty = y_max < 0.0
    y0 = jnp.where(is_empty, 0.0, y_min)
    x0 = jnp.where(is_empty, 0.0, x_min)
    y1 = jnp.where(is_empty, 1.0, y_max)
    x1 = jnp.where(is_empty, 1.0, x_max)
    return y0, x0, y1, x1


def _penalty_scalar(py0, px0, py1, px1, ty0, tx0, ty1, tx1):
    pred_area = (py1 - py0 + 1.0) * (px1 - px0 + 1.0)
    true_area = (ty1 - ty0 + 1.0) * (tx1 - tx0 + 1.0)
    area_penalty = jnp.maximum(pred_area - true_area, 0.0) / (true_area + 1.0)
    dy = (py0 + py1) / 2.0 - (ty0 + ty1) / 2.0
    dx = (px0 + px1) / 2.0 - (tx0 + tx1) / 2.0
    center_offset = jnp.sqrt(dy * dy + dx * dx) / 20.0
    return area_penalty + center_offset


def _body(p_ref, t_ref, out_ref, colp_ref, colt_ref, acc_ref):
    b = pl.program_id(0)
    c = pl.program_id(1)
    first = c == 0

    m_p = jnp.max(p_ref[0], axis=0)
    m_t = jnp.max(t_ref[0], axis=0)

    cp = jnp.max(m_p, axis=0, keepdims=True)
    ct = jnp.max(m_t, axis=0, keepdims=True)
    colp_ref[...] = jnp.where(first, cp, jnp.maximum(colp_ref[...], cp))
    colt_ref[...] = jnp.where(first, ct, jnp.maximum(colt_ref[...], ct))

    rp = jnp.max(m_p, axis=1, keepdims=True) > _THRESHOLD
    rt = jnp.max(m_t, axis=1, keepdims=True) > _TRUE_THRESHOLD
    yidx = jax.lax.broadcasted_iota(jnp.int32, (_RC, 1), 0).astype(
        jnp.float32) + (c * _RC).astype(jnp.float32)
    ymin_p = jnp.min(jnp.where(rp, yidx, float(_H)))
    ymax_p = jnp.max(jnp.where(rp, yidx, -1.0))
    ymin_t = jnp.min(jnp.where(rt, yidx, float(_H)))
    ymax_t = jnp.max(jnp.where(rt, yidx, -1.0))

    acc_ref[0] = jnp.where(first, ymin_p, jnp.minimum(acc_ref[0], ymin_p))
    acc_ref[1] = jnp.where(first, ymax_p, jnp.maximum(acc_ref[1], ymax_p))
    acc_ref[2] = jnp.where(first, ymin_t, jnp.minimum(acc_ref[2], ymin_t))
    acc_ref[3] = jnp.where(first, ymax_t, jnp.maximum(acc_ref[3], ymax_t))

    @pl.when(c == _NCH - 1)
    def _finish_image():
        xmap = jax.lax.broadcasted_iota(jnp.int32, (1, _W), 1).astype(
            jnp.float32)
        cmp_ = colp_ref[...] > _THRESHOLD
        cmt_ = colt_ref[...] > _TRUE_THRESHOLD
        xmin_p = jnp.min(jnp.where(cmp_, xmap, float(_W)))
        xmax_p = jnp.max(jnp.where(cmp_, xmap, -1.0))
        xmin_t = jnp.min(jnp.where(cmt_, xmap, float(_W)))
        xmax_t = jnp.max(jnp.where(cmt_, xmap, -1.0))

        pb = _bbox_from_scalars(acc_ref[0], acc_ref[1], xmin_p, xmax_p)
        tb = _bbox_from_scalars(acc_ref[2], acc_ref[3], xmin_t, xmax_t)
        pen = _penalty_scalar(*pb, *tb)
        psum = jnp.where(b == 0, pen, acc_ref[4] + pen)
        acc_ref[4] = psum

        @pl.when(b == _B - 1)
        def _write_out():
            out_ref[0, 0] = _PENALTY_WEIGHT * psum / float(_B)


@jax.jit
def _run(pred_t, true_t):
    out = pl.pallas_call(
        _body,
        grid=(_B, _NCH),
        in_specs=[
            pl.BlockSpec((1, _C, _RC, _W), lambda b, c: (b, 0, c, 0)),
            pl.BlockSpec((1, _C, _RC, _W), lambda b, c: (b, 0, c, 0)),
        ],
        out_specs=pl.BlockSpec(memory_space=pltpu.SMEM),
        out_shape=jax.ShapeDtypeStruct((1, 1), jnp.float32),
        scratch_shapes=[
            pltpu.VMEM((1, _W), jnp.float32),
            pltpu.VMEM((1, _W), jnp.float32),
            pltpu.SMEM((8,), jnp.float32),
        ],
        compiler_params=pltpu.CompilerParams(
            dimension_semantics=("arbitrary", "arbitrary"),
        ),
    )(pred_t, true_t)
    return out[0, 0]


def kernel(prediction_probs, expected_onehot):
    pred_t = prediction_probs.transpose(0, 3, 1, 2)
    true_t = expected_onehot.transpose(0, 3, 1, 2)
    return _run(pred_t, true_t)


# speedup vs baseline: 17.2610x; 1.0208x over previous
import jax
import jax.numpy as jnp
from jax.experimental import pallas as pl
from jax.experimental.pallas import tpu as pltpu

_THRESHOLD = 0.3
_TRUE_THRESHOLD = 0.5
_PENALTY_WEIGHT = 0.05

_B, _H, _W, _C = 8, 512, 512, 21
_RC = 128
_NCH = _H // _RC


def _bbox_from_scalars(y_min, y_max, x_min, x_max):
    is_empty = y_max < 0.0
    y0 = jnp.where(is_empty, 0.0, y_min)
    x0 = jnp.where(is_empty, 0.0, x_min)
    y1 = jnp.where(is_empty, 1.0, y_max)
    x1 = jnp.where(is_empty, 1.0, x_max)
    return y0, x0, y1, x1


def _penalty_scalar(py0, px0, py1, px1, ty0, tx0, ty1, tx1):
    pred_area = (py1 - py0 + 1.0) * (px1 - px0 + 1.0)
    true_area = (ty1 - ty0 + 1.0) * (tx1 - tx0 + 1.0)
    area_penalty = jnp.maximum(pred_area - true_area, 0.0) / (true_area + 1.0)
    dy = (py0 + py1) / 2.0 - (ty0 + ty1) / 2.0
    dx = (px0 + px1) / 2.0 - (tx0 + tx1) / 2.0
    center_offset = jnp.sqrt(dy * dy + dx * dx) / 20.0
    return area_penalty + center_offset


def _body(p_ref, t_ref, out_ref, colp_ref, colt_ref, acc_ref):
    b = pl.program_id(0)
    c = pl.program_id(1)
    first = c == 0

    m_p = jnp.max(p_ref[0], axis=0)
    m_t = jnp.max(t_ref[0], axis=0)

    cp = jnp.max(m_p, axis=0, keepdims=True)
    ct = jnp.max(m_t, axis=0, keepdims=True)
    colp_ref[...] = jnp.where(first, cp, jnp.maximum(colp_ref[...], cp))
    colt_ref[...] = jnp.where(first, ct, jnp.maximum(colt_ref[...], ct))

    rp = jnp.max(m_p, axis=1, keepdims=True) > _THRESHOLD
    rt = jnp.max(m_t, axis=1, keepdims=True) > _TRUE_THRESHOLD
    yidx = jax.lax.broadcasted_iota(jnp.int32, (_RC, 1), 0).astype(
        jnp.float32) + (c * _RC).astype(jnp.float32)
    ymin_p = jnp.min(jnp.where(rp, yidx, float(_H)))
    ymax_p = jnp.max(jnp.where(rp, yidx, -1.0))
    ymin_t = jnp.min(jnp.where(rt, yidx, float(_H)))
    ymax_t = jnp.max(jnp.where(rt, yidx, -1.0))

    acc_ref[0] = jnp.where(first, ymin_p, jnp.minimum(acc_ref[0], ymin_p))
    acc_ref[1] = jnp.where(first, ymax_p, jnp.maximum(acc_ref[1], ymax_p))
    acc_ref[2] = jnp.where(first, ymin_t, jnp.minimum(acc_ref[2], ymin_t))
    acc_ref[3] = jnp.where(first, ymax_t, jnp.maximum(acc_ref[3], ymax_t))

    @pl.when(c == _NCH - 1)
    def _finish_image():
        xmap = jax.lax.broadcasted_iota(jnp.int32, (1, _W), 1).astype(
            jnp.float32)
        cmp_ = colp_ref[...] > _THRESHOLD
        cmt_ = colt_ref[...] > _TRUE_THRESHOLD
        xmin_p = jnp.min(jnp.where(cmp_, xmap, float(_W)))
        xmax_p = jnp.max(jnp.where(cmp_, xmap, -1.0))
        xmin_t = jnp.min(jnp.where(cmt_, xmap, float(_W)))
        xmax_t = jnp.max(jnp.where(cmt_, xmap, -1.0))

        pb = _bbox_from_scalars(acc_ref[0], acc_ref[1], xmin_p, xmax_p)
        tb = _bbox_from_scalars(acc_ref[2], acc_ref[3], xmin_t, xmax_t)
        pen = _penalty_scalar(*pb, *tb)
        psum = jnp.where(b == 0, pen, acc_ref[4] + pen)
        acc_ref[4] = psum

        @pl.when(b == _B - 1)
        def _write_out():
            out_ref[0, 0] = _PENALTY_WEIGHT * psum / float(_B)


@jax.jit
def _run(pred_t, true_t):
    out = pl.pallas_call(
        _body,
        grid=(_B, _NCH),
        in_specs=[
            pl.BlockSpec((1, _C, _RC, _W), lambda b, c: (b, 0, c, 0)),
            pl.BlockSpec((1, _C, _RC, _W), lambda b, c: (b, 0, c, 0)),
        ],
        out_specs=pl.BlockSpec(memory_space=pltpu.SMEM),
        out_shape=jax.ShapeDtypeStruct((1, 1), jnp.float32),
        scratch_shapes=[
            pltpu.VMEM((1, _W), jnp.float32),
            pltpu.VMEM((1, _W), jnp.float32),
            pltpu.SMEM((8,), jnp.float32),
        ],
        compiler_params=pltpu.CompilerParams(
            dimension_semantics=("arbitrary", "arbitrary"),
        ),
    )(pred_t, true_t)
    return out[0, 0]


def kernel(prediction_probs, expected_onehot):
    pred_t = prediction_probs.transpose(0, 3, 1, 2)
    true_t = expected_onehot.transpose(0, 3, 1, 2)
    return _run(pred_t, true_t)
